# Initial kernel scaffold; baseline (speedup 1.0000x reference)
import jax
import jax.numpy as jnp
from jax.experimental import pallas as pl


def kernel(input, adj_low, adj_high, weight_low, weight_high, weight_mlp, att_vec_low, att_vec_high, att_vec_mlp, att_vec):
    raise NotImplementedError("write your pallas kernel here")


# speedup vs baseline: 1.0145x; 1.0145x over previous
import functools

import jax
import jax.numpy as jnp
from jax.experimental import pallas as pl

N = 10000
D = 128
BM = 200


def _proj_kernel(x_ref, wl_ref, wh_ref, wm_ref, avm_ref,
                 u_ref, v_ref, m_ref, lm_ref):
    x = x_ref[...]
    u_ref[...] = jnp.dot(x, wl_ref[...], preferred_element_type=jnp.float32)
    v_ref[...] = jnp.dot(x, wh_ref[...], preferred_element_type=jnp.float32)
    m = jnp.maximum(jnp.dot(x, wm_ref[...], preferred_element_type=jnp.float32), 0.0)
    m_ref[...] = m
    lm_ref[...] = jnp.dot(m, avm_ref[...], preferred_element_type=jnp.float32)


def _combine_kernel(adj_l_ref, adj_h_ref, u_ref, v_ref, m_ref, lm_ref,
                    avl_ref, avh_ref, att_ref, out_ref):
    ol = jnp.maximum(
        jnp.dot(adj_l_ref[...], u_ref[...], preferred_element_type=jnp.float32), 0.0)
    oh = jnp.maximum(
        jnp.dot(adj_h_ref[...], v_ref[...], preferred_element_type=jnp.float32), 0.0)
    m = m_ref[...]
    ll = jnp.dot(ol, avl_ref[...], preferred_element_type=jnp.float32)
    lh = jnp.dot(oh, avh_ref[...], preferred_element_type=jnp.float32)
    logits = jnp.concatenate([ll, lh, lm_ref[...]], axis=1)
    z = jnp.dot(jax.nn.sigmoid(logits), att_ref[...],
                preferred_element_type=jnp.float32) * (1.0 / 3.0)
    zmax = jnp.max(z, axis=1, keepdims=True)
    e = jnp.exp(z - zmax)
    att = e / jnp.sum(e, axis=1, keepdims=True)
    out_ref[...] = 3.0 * (att[:, 0:1] * ol + att[:, 1:2] * oh + att[:, 2:3] * m)


@jax.jit
def kernel(input, adj_low, adj_high, weight_low, weight_high, weight_mlp,
           att_vec_low, att_vec_high, att_vec_mlp, att_vec):
    u, v, m, lm = pl.pallas_call(
        _proj_kernel,
        out_shape=(
            jax.ShapeDtypeStruct((N, D), jnp.float32),
            jax.ShapeDtypeStruct((N, D), jnp.float32),
            jax.ShapeDtypeStruct((N, D), jnp.float32),
            jax.ShapeDtypeStruct((N, 1), jnp.float32),
        ),
    )(input, weight_low, weight_high, weight_mlp, att_vec_mlp)

    nb = N // BM
    out = pl.pallas_call(
        _combine_kernel,
        grid=(nb,),
        in_specs=[
            pl.BlockSpec((BM, N), lambda i: (i, 0)),
            pl.BlockSpec((BM, N), lambda i: (i, 0)),
            pl.BlockSpec((N, D), lambda i: (0, 0)),
            pl.BlockSpec((N, D), lambda i: (0, 0)),
            pl.BlockSpec((BM, D), lambda i: (i, 0)),
            pl.BlockSpec((BM, 1), lambda i: (i, 0)),
            pl.BlockSpec((D, 1), lambda i: (0, 0)),
            pl.BlockSpec((D, 1), lambda i: (0, 0)),
            pl.BlockSpec((3, 3), lambda i: (0, 0)),
        ],
        out_specs=pl.BlockSpec((BM, D), lambda i: (i, 0)),
        out_shape=jax.ShapeDtypeStruct((N, D), jnp.float32),
    )(adj_low, adj_high, u, v, m, lm, att_vec_low, att_vec_high, att_vec)
    return out


# speedup vs baseline: 1.0184x; 1.0038x over previous
import functools

import jax
import jax.numpy as jnp
from jax.experimental import pallas as pl

N = 10000
D = 128
BM = 200


def _proj_kernel(x_ref, wl_ref, wh_ref, wm_ref, avm_ref,
                 u_ref, v_ref, m_ref, lm_ref):
    x = x_ref[...]
    u_ref[...] = jnp.dot(x, wl_ref[...],
                         preferred_element_type=jnp.float32).astype(jnp.bfloat16)
    v_ref[...] = jnp.dot(x, wh_ref[...],
                         preferred_element_type=jnp.float32).astype(jnp.bfloat16)
    m = jnp.maximum(jnp.dot(x, wm_ref[...], preferred_element_type=jnp.float32), 0.0)
    m_ref[...] = m
    lm_ref[...] = jnp.dot(m, avm_ref[...], preferred_element_type=jnp.float32)


def _combine_kernel(adj_l_ref, adj_h_ref, u_ref, v_ref, m_ref, lm_ref,
                    avl_ref, avh_ref, att_ref, out_ref):
    ol = jnp.maximum(
        jnp.dot(adj_l_ref[...].astype(jnp.bfloat16), u_ref[...],
                preferred_element_type=jnp.float32), 0.0)
    oh = jnp.maximum(
        jnp.dot(adj_h_ref[...].astype(jnp.bfloat16), v_ref[...],
                preferred_element_type=jnp.float32), 0.0)
    m = m_ref[...]
    ll = jnp.dot(ol, avl_ref[...], preferred_element_type=jnp.float32)
    lh = jnp.dot(oh, avh_ref[...], preferred_element_type=jnp.float32)
    logits = jnp.concatenate([ll, lh, lm_ref[...]], axis=1)
    z = jnp.dot(jax.nn.sigmoid(logits), att_ref[...],
                preferred_element_type=jnp.float32) * (1.0 / 3.0)
    zmax = jnp.max(z, axis=1, keepdims=True)
    e = jnp.exp(z - zmax)
    att = e / jnp.sum(e, axis=1, keepdims=True)
    out_ref[...] = 3.0 * (att[:, 0:1] * ol + att[:, 1:2] * oh + att[:, 2:3] * m)


@jax.jit
def kernel(input, adj_low, adj_high, weight_low, weight_high, weight_mlp,
           att_vec_low, att_vec_high, att_vec_mlp, att_vec):
    u, v, m, lm = pl.pallas_call(
        _proj_kernel,
        out_shape=(
            jax.ShapeDtypeStruct((N, D), jnp.bfloat16),
            jax.ShapeDtypeStruct((N, D), jnp.bfloat16),
            jax.ShapeDtypeStruct((N, D), jnp.float32),
            jax.ShapeDtypeStruct((N, 1), jnp.float32),
        ),
    )(input, weight_low, weight_high, weight_mlp, att_vec_mlp)

    nb = N // BM
    out = pl.pallas_call(
        _combine_kernel,
        grid=(nb,),
        in_specs=[
            pl.BlockSpec((BM, N), lambda i: (i, 0)),
            pl.BlockSpec((BM, N), lambda i: (i, 0)),
            pl.BlockSpec((N, D), lambda i: (0, 0)),
            pl.BlockSpec((N, D), lambda i: (0, 0)),
            pl.BlockSpec((BM, D), lambda i: (i, 0)),
            pl.BlockSpec((BM, 1), lambda i: (i, 0)),
            pl.BlockSpec((D, 1), lambda i: (0, 0)),
            pl.BlockSpec((D, 1), lambda i: (0, 0)),
            pl.BlockSpec((3, 3), lambda i: (0, 0)),
        ],
        out_specs=pl.BlockSpec((BM, D), lambda i: (i, 0)),
        out_shape=jax.ShapeDtypeStruct((N, D), jnp.float32),
    )(adj_low, adj_high, u, v, m, lm, att_vec_low, att_vec_high, att_vec)
    return out


# speedup vs baseline: 1.0809x; 1.0614x over previous
import jax
import jax.numpy as jnp
from jax.experimental import pallas as pl
from jax.experimental import pallas
import jax.experimental.pallas.tpu as pltpu

N = 10000
D = 128
BM = 200


def _fused_kernel(adj_l_ref, adj_h_ref, x_ref, wl_ref, wh_ref, wm_ref,
                  avl_ref, avh_ref, avm_ref, att_ref, out_ref,
                  u_s, v_s):
    i = pl.program_id(0)

    @pl.when(i == 0)
    def _init():
        xb = x_ref[...].astype(jnp.bfloat16)
        u_s[...] = jnp.dot(xb, wl_ref[...].astype(jnp.bfloat16),
                           preferred_element_type=jnp.float32).astype(jnp.bfloat16)
        v_s[...] = jnp.dot(xb, wh_ref[...].astype(jnp.bfloat16),
                           preferred_element_type=jnp.float32).astype(jnp.bfloat16)

    ol = jnp.maximum(
        jnp.dot(adj_l_ref[...].astype(jnp.bfloat16), u_s[...],
                preferred_element_type=jnp.float32), 0.0)
    oh = jnp.maximum(
        jnp.dot(adj_h_ref[...].astype(jnp.bfloat16), v_s[...],
                preferred_element_type=jnp.float32), 0.0)
    x_blk = x_ref[pl.ds(i * BM, BM), :].astype(jnp.bfloat16)
    m = jnp.maximum(
        jnp.dot(x_blk, wm_ref[...].astype(jnp.bfloat16),
                preferred_element_type=jnp.float32), 0.0)
    ll = jnp.dot(ol, avl_ref[...], preferred_element_type=jnp.float32)
    lh = jnp.dot(oh, avh_ref[...], preferred_element_type=jnp.float32)
    lm = jnp.dot(m, avm_ref[...], preferred_element_type=jnp.float32)
    logits = jnp.concatenate([ll, lh, lm], axis=1)
    z = jnp.dot(jax.nn.sigmoid(logits), att_ref[...],
                preferred_element_type=jnp.float32) * (1.0 / 3.0)
    zmax = jnp.max(z, axis=1, keepdims=True)
    e = jnp.exp(z - zmax)
    att = e / jnp.sum(e, axis=1, keepdims=True)
    out_ref[...] = 3.0 * (att[:, 0:1] * ol + att[:, 1:2] * oh + att[:, 2:3] * m)


@jax.jit
def kernel(input, adj_low, adj_high, weight_low, weight_high, weight_mlp,
           att_vec_low, att_vec_high, att_vec_mlp, att_vec):
    nb = N // BM
    out = pl.pallas_call(
        _fused_kernel,
        grid=(nb,),
        in_specs=[
            pl.BlockSpec((BM, N), lambda i: (i, 0)),
            pl.BlockSpec((BM, N), lambda i: (i, 0)),
            pl.BlockSpec((N, D), lambda i: (0, 0)),
            pl.BlockSpec((D, D), lambda i: (0, 0)),
            pl.BlockSpec((D, D), lambda i: (0, 0)),
            pl.BlockSpec((D, D), lambda i: (0, 0)),
            pl.BlockSpec((D, 1), lambda i: (0, 0)),
            pl.BlockSpec((D, 1), lambda i: (0, 0)),
            pl.BlockSpec((D, 1), lambda i: (0, 0)),
            pl.BlockSpec((3, 3), lambda i: (0, 0)),
        ],
        out_specs=pl.BlockSpec((BM, D), lambda i: (i, 0)),
        out_shape=jax.ShapeDtypeStruct((N, D), jnp.float32),
        scratch_shapes=[
            pltpu.VMEM((N, D), jnp.bfloat16),
            pltpu.VMEM((N, D), jnp.bfloat16),
        ],
    )(adj_low, adj_high, input, weight_low, weight_high, weight_mlp,
      att_vec_low, att_vec_high, att_vec_mlp, att_vec)
    return out
